# baseline (speedup 1.0000x reference)
import functools

import jax
import jax.numpy as jnp
from jax import lax
from jax.experimental import pallas as pl
from jax.experimental.pallas import tpu as pltpu
from jax.experimental.pallas import tpu_sc as plsc

WIDTH = 1024
HEIGHT = 1024
NBATCH = 16
NPTS = 42
TOTAL = NBATCH * NPTS
LANES = 16
NCHUNK = TOTAL // LANES

_mesh = plsc.VectorSubcoreMesh(core_axis_name="c", subcore_axis_name="s")


@functools.partial(
    pl.kernel,
    out_type=jax.ShapeDtypeStruct((LANES,), jnp.float32),
    mesh=_mesh,
    scratch_types=[
        pltpu.VMEM((2 * TOTAL,), jnp.float32),
        pltpu.VMEM((TOTAL,), jnp.float32),
        pltpu.VMEM((TOTAL,), jnp.float32),
        pltpu.VMEM((LANES,), jnp.float32),
        pltpu.SemaphoreType.DMA,
    ],
)
def _mask_loss_sc(pred_hbm, dist_hbm, out_hbm, pred_v, vals_v, w_v, out_v, sem):
    cid = lax.axis_index("c")
    sid = lax.axis_index("s")

    @pl.when(jnp.logical_and(cid == 0, sid == 0))
    def _():
        pltpu.sync_copy(pred_hbm, pred_v)
        lane = lax.iota(jnp.int32, LANES)
        copies = []
        for c in range(NCHUNK):
            px = pred_v[pl.ds(c * LANES, LANES)]
            py = pred_v[pl.ds(TOTAL + c * LANES, LANES)]
            xi = ((px + 1.0) * (WIDTH * 0.5)).astype(jnp.int32)
            yi = ((py + 1.0) * (HEIGHT * 0.5)).astype(jnp.int32)
            b_lo = (c * LANES) // NPTS
            b_hi = (c * LANES + LANES - 1) // NPTS
            p = c * LANES + lane
            b = jnp.where(p >= b_hi * NPTS, b_hi, b_lo).astype(jnp.int32)
            flat = b * (WIDTH * HEIGHT) + xi * HEIGHT + yi
            cp = pltpu.make_async_copy(
                dist_hbm.at[flat], vals_v.at[pl.ds(c * LANES, LANES)], sem
            )
            cp.start()
            copies.append(cp)
            qx = px / px
            qy = py / py
            tx = jnp.where(qx >= 1.0, jnp.float32(1.0), 0.0 * qx)
            ty = jnp.where(qy >= 1.0, jnp.float32(1.0), 0.0 * qy)
            w_v[pl.ds(c * LANES, LANES)] = tx * ty
        for cp in copies:
            cp.wait()
        acc = jnp.zeros((LANES,), jnp.float32)
        for c in range(NCHUNK):
            sl = pl.ds(c * LANES, LANES)
            acc = acc + vals_v[sl] * w_v[sl]
        total = acc[0]
        for i in range(1, LANES):
            total = total + acc[i]
        out_v[...] = jnp.full((LANES,), total * (1.0 / TOTAL), jnp.float32)
        pltpu.sync_copy(out_v, out_hbm)


def kernel(predict, distmap):
    pred_sep = predict.reshape(-1, 2).T.reshape(-1)
    dist_flat = distmap.reshape(-1)
    out = _mask_loss_sc(pred_sep, dist_flat)
    return out[0]


# speedup vs baseline: 2.6708x; 2.6708x over previous
import functools

import jax
import jax.numpy as jnp
from jax import lax
from jax.experimental import pallas as pl
from jax.experimental.pallas import tpu as pltpu
from jax.experimental.pallas import tpu_sc as plsc

WIDTH = 1024
HEIGHT = 1024
NBATCH = 16
NPTS = 42
TOTAL = NBATCH * NPTS
LANES = 16
NCHUNK = TOTAL // LANES
NTILES = 16
NSLOTS = 3
PADPTS = NTILES * NSLOTS * LANES

_mesh = plsc.VectorSubcoreMesh(core_axis_name="c", subcore_axis_name="s")


@functools.partial(
    pl.kernel,
    out_type=jax.ShapeDtypeStruct((LANES,), jnp.float32),
    mesh=_mesh,
    scratch_types=[
        pltpu.VMEM((2 * PADPTS,), jnp.float32),
        pltpu.VMEM((PADPTS,), jnp.int32),
        pltpu.VMEM((NSLOTS * LANES + 1, WIDTH), jnp.float32),
        pltpu.VMEM((NTILES, LANES), jnp.float32),
        pltpu.VMEM((LANES,), jnp.float32),
        pltpu.HBM((NTILES, LANES), jnp.float32),
        pltpu.SemaphoreType.DMA,
    ],
)
def _mask_loss_sc(pred_hbm, boffs_hbm, dist_hbm, out_hbm,
                  pred_v, boffs_v, rows_v, part_v, out_v, parts_hbm, sem):
    cid = lax.axis_index("c")
    sid = lax.axis_index("s")

    @pl.when(cid == 0)
    def _():
        pltpu.sync_copy(pred_hbm, pred_v)
        pltpu.sync_copy(boffs_hbm, boffs_v)
        lane = lax.iota(jnp.int32, LANES)
        zero = jnp.zeros((LANES,), jnp.float32)

        copies, ws, yis = [], [], []
        for l in range(NSLOTS):
            c = sid + l * NTILES
            base = c * LANES
            px = pred_v[pl.ds(base, LANES)]
            py = pred_v[pl.ds(PADPTS + base, LANES)]
            xi = ((px + 1.0) * (WIDTH * 0.5)).astype(jnp.int32)
            yi = ((py + 1.0) * (HEIGHT * 0.5)).astype(jnp.int32)
            xi = jnp.minimum(xi, WIDTH - 1)
            yi = jnp.minimum(yi, HEIGHT - 1)
            row = boffs_v[pl.ds(base, LANES)] + xi
            cp = pltpu.make_async_copy(
                dist_hbm.at[row], rows_v.at[pl.ds(l * LANES, LANES)], sem
            )
            cp.start()
            copies.append(cp)
            qx = px / px
            qy = py / py
            tx = jnp.where(qx >= 1.0, jnp.float32(1.0), 0.0 * qx)
            ty = jnp.where(qy >= 1.0, jnp.float32(1.0), 0.0 * qy)
            w = tx * ty
            valid = jnp.where(
                jnp.full((LANES,), c, jnp.int32) < NCHUNK, 1.0, 0.0
            ).astype(jnp.float32)
            ws.append(w * valid)
            yis.append(yi)

        acc = zero
        for l in range(NSLOTS):
            copies[l].wait()
            w, yi = ws[l], yis[l]
            for j in range(LANES):
                yj = yi[j]
                ybase = pl.multiple_of(jnp.bitwise_and(yj, jnp.int32(-16)), 16)
                d = jnp.bitwise_and(yj, jnp.int32(15))
                v = rows_v[l * LANES + j, pl.ds(ybase, LANES)]
                sel = lane == jnp.full((LANES,), d, jnp.int32)
                acc = acc + jnp.where(
                    sel, v * jnp.full((LANES,), w[j], jnp.float32), zero
                )

        out_v[...] = acc
        pltpu.sync_copy(out_v, parts_hbm.at[sid])
        plsc.subcore_barrier()

        @pl.when(sid == 0)
        def _():
            pltpu.sync_copy(parts_hbm, part_v)
            tot_v = zero
            for t in range(NTILES):
                tot_v = tot_v + part_v[t]
            total = tot_v[0]
            for i in range(1, LANES):
                total = total + tot_v[i]
            out_v[...] = jnp.full((LANES,), total * (1.0 / TOTAL), jnp.float32)
            pltpu.sync_copy(out_v, out_hbm)


def kernel(predict, distmap):
    p = predict.reshape(-1, 2)
    pad = jnp.full((PADPTS - TOTAL,), 0.5, jnp.float32)
    pred_sep = jnp.concatenate([p[:, 0], pad, p[:, 1], pad])
    boffs = jnp.concatenate([
        (jnp.arange(TOTAL, dtype=jnp.int32) // NPTS) * WIDTH,
        jnp.zeros((PADPTS - TOTAL,), jnp.int32),
    ])
    dist2 = distmap.reshape(NBATCH * WIDTH, HEIGHT)
    out = _mask_loss_sc(pred_sep, boffs, dist2)
    return out[0]


# speedup vs baseline: 2.9986x; 1.1228x over previous
import functools

import jax
import jax.numpy as jnp
import numpy as np
from jax import lax
from jax.experimental import pallas as pl
from jax.experimental.pallas import tpu as pltpu
from jax.experimental.pallas import tpu_sc as plsc

WIDTH = 1024
HEIGHT = 1024
NBATCH = 16
NPTS = 42
TOTAL = NBATCH * NPTS
LANES = 16
NCHUNK = TOTAL // LANES
NTILES = 16
NSLOTS = 3
PADPTS = NTILES * NSLOTS * LANES

_BOFFS = np.concatenate([
    (np.arange(TOTAL, dtype=np.int32) // NPTS) * WIDTH,
    np.zeros(PADPTS - TOTAL, np.int32),
])
_PRED_PAD = np.full(2 * (PADPTS - TOTAL), 0.5, np.float32)

_mesh = plsc.VectorSubcoreMesh(core_axis_name="c", subcore_axis_name="s")


@functools.partial(
    pl.kernel,
    out_type=jax.ShapeDtypeStruct((LANES,), jnp.float32),
    mesh=_mesh,
    scratch_types=[
        pltpu.VMEM((2 * PADPTS,), jnp.float32),
        pltpu.VMEM((PADPTS,), jnp.int32),
        pltpu.VMEM((NSLOTS * LANES, LANES), jnp.float32),
        pltpu.VMEM((NTILES, LANES), jnp.float32),
        pltpu.VMEM((LANES,), jnp.float32),
        pltpu.HBM((NTILES, LANES), jnp.float32),
        pltpu.SemaphoreType.DMA,
    ],
)
def _mask_loss_sc(pred_hbm, boffs_hbm, dist_hbm, out_hbm,
                  pred_v, boffs_v, seg_v, part_v, out_v, parts_hbm, sem):
    cid = lax.axis_index("c")
    sid = lax.axis_index("s")

    @pl.when(cid == 0)
    def _():
        pltpu.sync_copy(pred_hbm, pred_v)
        pltpu.sync_copy(boffs_hbm, boffs_v)
        lane = lax.iota(jnp.int32, LANES)
        zero = jnp.zeros((LANES,), jnp.float32)

        copies, ts, iis = [], [], []
        for l in range(NSLOTS):
            c = sid + l * NTILES
            valid = (c < NCHUNK).astype(jnp.float32)
            boffs = boffs_v[pl.ds(c * LANES, LANES)]
            for h in range(2):
                v = pred_v[pl.ds(c * 2 * LANES + h * LANES, LANES)]
                ii = jnp.minimum(
                    ((v + 1.0) * (WIDTH * 0.5)).astype(jnp.int32), WIDTH - 1
                )
                q = v / v
                t = jnp.where(q >= 1.0, jnp.float32(1.0), 0.0 * q)
                t = t * jnp.full((LANES,), valid, jnp.float32)
                ts.append(t)
                iis.append(ii)
                for j in range(LANES // 2):
                    p_local = l * LANES + h * (LANES // 2) + j
                    row = boffs[h * (LANES // 2) + j] + ii[2 * j]
                    yj = ii[2 * j + 1]
                    ybase = pl.multiple_of(
                        jnp.bitwise_and(yj, jnp.int32(-16)), 16
                    )
                    cp = pltpu.make_async_copy(
                        dist_hbm.at[row, pl.ds(ybase, LANES)],
                        seg_v.at[p_local],
                        sem,
                    )
                    cp.start()
                    copies.append(cp)

        for cp in copies:
            cp.wait()
        acc = zero
        for l in range(NSLOTS):
            for h in range(2):
                t = ts[2 * l + h]
                ii = iis[2 * l + h]
                for j in range(LANES // 2):
                    p_local = l * LANES + h * (LANES // 2) + j
                    w = t[2 * j] * t[2 * j + 1]
                    d = jnp.bitwise_and(ii[2 * j + 1], jnp.int32(15))
                    seg = seg_v[p_local]
                    sel = lane == jnp.full((LANES,), d, jnp.int32)
                    acc = acc + jnp.where(
                        sel, seg * jnp.full((LANES,), w, jnp.float32), zero
                    )

        out_v[...] = acc
        pltpu.sync_copy(out_v, parts_hbm.at[sid])
        plsc.subcore_barrier()

        @pl.when(sid == 0)
        def _():
            pltpu.sync_copy(parts_hbm, part_v)
            tot_v = zero
            for t in range(NTILES):
                tot_v = tot_v + part_v[t]
            total = tot_v[0]
            for i in range(1, LANES):
                total = total + tot_v[i]
            out_v[...] = jnp.full((LANES,), total * (1.0 / TOTAL), jnp.float32)
            pltpu.sync_copy(out_v, out_hbm)


def kernel(predict, distmap):
    pred = jnp.concatenate([predict.reshape(-1), jnp.asarray(_PRED_PAD)])
    out = _mask_loss_sc(pred, jnp.asarray(_BOFFS),
                        distmap.reshape(NBATCH * WIDTH, HEIGHT))
    return out[0]
